# baseline (speedup 1.0000x reference)
import functools

import jax
import jax.numpy as jnp
from jax.experimental import pallas as pl
from jax.experimental.pallas import tpu as pltpu

_NBUF = 16


def _first_argmax(z, axis):
    zmax = jnp.max(z, axis=axis, keepdims=True)
    iota = jax.lax.broadcasted_iota(jnp.int32, z.shape, axis)
    big = jnp.int32(z.shape[axis])
    return jnp.min(jnp.where(z >= zmax, iota, big), axis=axis, keepdims=True)


def _routing(cat_ref, gam_ref, bet_ref, dep_ref, ua_ref, ud_ref, k):
    g = -jnp.log(-jnp.log(ua_ref[...]))
    z = cat_ref[...][None, :, :] + g
    idx = _first_argmax(z, axis=2)
    iota = jax.lax.broadcasted_iota(jnp.int32, z.shape, 2)
    oh = (iota == idx).astype(jnp.float32)
    s = jnp.sum(oh * gam_ref[...][None, :, :], axis=2)
    t = jnp.sum(oh * bet_ref[...][None, :, :], axis=2)

    gd = -jnp.log(-jnp.log(ud_ref[...]))
    zd = dep_ref[...] + gd
    m = _first_argmax(zd, axis=1)

    A = jnp.ones_like(m, dtype=jnp.float32)
    C = jnp.zeros_like(A)
    S = jnp.where(m == 0, A, 0.0)
    T = jnp.zeros_like(A)
    for i in range(k):
        si = s[:, i:i + 1]
        ti = t[:, i:i + 1]
        A = si * A
        C = si * C + ti
        S = jnp.where(m == i + 1, A, S)
        T = jnp.where(m == i + 1, C, T)
    return S, T


def _body(cat_ref, gam_ref, bet_ref, dep_ref, ua_ref, ud_ref, x_hbm, o_hbm,
          ibuf, obuf, in_sems, out_sems, *, k, B):
    for j in range(_NBUF):
        pltpu.make_async_copy(x_hbm.at[j], ibuf.at[j], in_sems.at[j]).start()

    S, T = _routing(cat_ref, gam_ref, bet_ref, dep_ref, ua_ref, ud_ref, k)

    for i in range(B):
        slot = i % _NBUF
        pltpu.make_async_copy(x_hbm.at[i], ibuf.at[slot], in_sems.at[slot]).wait()
        if i >= _NBUF:
            pltpu.make_async_copy(
                obuf.at[slot], o_hbm.at[i - _NBUF], out_sems.at[slot]).wait()
        obuf[slot] = S[i, 0] * ibuf[slot] + T[i, 0]
        pltpu.make_async_copy(obuf.at[slot], o_hbm.at[i], out_sems.at[slot]).start()
        nxt = i + _NBUF
        if nxt < B:
            pltpu.make_async_copy(
                x_hbm.at[nxt], ibuf.at[slot], in_sems.at[slot]).start()
    for i in range(B - _NBUF, B):
        slot = i % _NBUF
        pltpu.make_async_copy(obuf.at[slot], o_hbm.at[i], out_sems.at[slot]).wait()


def kernel(input, cat_logits, gammas, betas_aug, depth_logits):
    B = input.shape[0]
    k, d = cat_logits.shape
    C, H, W = input.shape[1:]

    key = jax.random.key(42)
    k_aug, k_depth = jax.random.split(key)
    ua = jax.random.uniform(k_aug, (B, k, d), minval=1e-6, maxval=1.0 - 1e-6)
    ud = jax.random.uniform(k_depth, (B, k + 1), minval=1e-6, maxval=1.0 - 1e-6)

    out = pl.pallas_call(
        functools.partial(_body, k=k, B=B),
        in_specs=[
            pl.BlockSpec(memory_space=pltpu.MemorySpace.VMEM),
            pl.BlockSpec(memory_space=pltpu.MemorySpace.VMEM),
            pl.BlockSpec(memory_space=pltpu.MemorySpace.VMEM),
            pl.BlockSpec(memory_space=pltpu.MemorySpace.VMEM),
            pl.BlockSpec(memory_space=pltpu.MemorySpace.VMEM),
            pl.BlockSpec(memory_space=pltpu.MemorySpace.VMEM),
            pl.BlockSpec(memory_space=pltpu.MemorySpace.HBM),
        ],
        out_specs=pl.BlockSpec(memory_space=pltpu.MemorySpace.HBM),
        out_shape=jax.ShapeDtypeStruct(input.shape, jnp.float32),
        scratch_shapes=[
            pltpu.VMEM((_NBUF, C, H, W), jnp.float32),
            pltpu.VMEM((_NBUF, C, H, W), jnp.float32),
            pltpu.SemaphoreType.DMA((_NBUF,)),
            pltpu.SemaphoreType.DMA((_NBUF,)),
        ],
    )(cat_logits, gammas, betas_aug, depth_logits.reshape(1, k + 1), ua, ud,
      input)
    return out


# speedup vs baseline: 1.1418x; 1.1418x over previous
import functools

import jax
import jax.numpy as jnp
from jax.experimental import pallas as pl
from jax.experimental.pallas import tpu as pltpu

_NBUF = 16


def _first_argmax(z, axis):
    zmax = jnp.max(z, axis=axis, keepdims=True)
    iota = jax.lax.broadcasted_iota(jnp.int32, z.shape, axis)
    big = jnp.int32(z.shape[axis])
    return jnp.min(jnp.where(z >= zmax, iota, big), axis=axis, keepdims=True)


def _routing(cat_ref, gam_ref, bet_ref, dep_ref, ua_ref, ud_ref, k):
    g = -jnp.log(-jnp.log(ua_ref[...]))
    z = cat_ref[...][None, :, :] + g
    idx = _first_argmax(z, axis=2)
    iota = jax.lax.broadcasted_iota(jnp.int32, z.shape, 2)
    oh = (iota == idx).astype(jnp.float32)
    s = jnp.sum(oh * gam_ref[...][None, :, :], axis=2)
    t = jnp.sum(oh * bet_ref[...][None, :, :], axis=2)

    gd = -jnp.log(-jnp.log(ud_ref[...]))
    zd = dep_ref[...] + gd
    m = _first_argmax(zd, axis=1)

    A = jnp.ones_like(m, dtype=jnp.float32)
    C = jnp.zeros_like(A)
    S = jnp.where(m == 0, A, 0.0)
    T = jnp.zeros_like(A)
    for i in range(k):
        si = s[:, i:i + 1]
        ti = t[:, i:i + 1]
        A = si * A
        C = si * C + ti
        S = jnp.where(m == i + 1, A, S)
        T = jnp.where(m == i + 1, C, T)
    return S, T


def _body(cat_ref, gam_ref, bet_ref, dep_ref, ua_ref, ud_ref, x_hbm, o_hbm,
          ibuf, obuf, in_sems, out_sems, *, k, B):
    for j in range(_NBUF):
        pltpu.make_async_copy(x_hbm.at[j], ibuf.at[j], in_sems.at[j]).start()

    S, T = _routing(cat_ref, gam_ref, bet_ref, dep_ref, ua_ref, ud_ref, k)

    for i in range(B):
        slot = i % _NBUF
        pltpu.make_async_copy(x_hbm.at[i], ibuf.at[slot], in_sems.at[slot]).wait()
        if i >= _NBUF:
            pltpu.make_async_copy(
                obuf.at[slot], o_hbm.at[i - _NBUF], out_sems.at[slot]).wait()
        obuf[slot] = S[i, 0] * ibuf[slot] + T[i, 0]
        pltpu.make_async_copy(obuf.at[slot], o_hbm.at[i], out_sems.at[slot]).start()
        nxt = i + _NBUF
        if nxt < B:
            pltpu.make_async_copy(
                x_hbm.at[nxt], ibuf.at[slot], in_sems.at[slot]).start()
    for i in range(B - _NBUF, B):
        slot = i % _NBUF
        pltpu.make_async_copy(obuf.at[slot], o_hbm.at[i], out_sems.at[slot]).wait()


def kernel(input, cat_logits, gammas, betas_aug, depth_logits):
    B = input.shape[0]
    k, d = cat_logits.shape
    C = input.shape[1]
    H, W = 392, 128
    x4 = input.reshape(B, C, H, W)

    key = jax.random.key(42)
    k_aug, k_depth = jax.random.split(key)
    ua = jax.random.uniform(k_aug, (B, k, d), minval=1e-6, maxval=1.0 - 1e-6)
    ud = jax.random.uniform(k_depth, (B, k + 1), minval=1e-6, maxval=1.0 - 1e-6)

    out = pl.pallas_call(
        functools.partial(_body, k=k, B=B),
        in_specs=[
            pl.BlockSpec(memory_space=pltpu.MemorySpace.VMEM),
            pl.BlockSpec(memory_space=pltpu.MemorySpace.VMEM),
            pl.BlockSpec(memory_space=pltpu.MemorySpace.VMEM),
            pl.BlockSpec(memory_space=pltpu.MemorySpace.VMEM),
            pl.BlockSpec(memory_space=pltpu.MemorySpace.VMEM),
            pl.BlockSpec(memory_space=pltpu.MemorySpace.VMEM),
            pl.BlockSpec(memory_space=pltpu.MemorySpace.HBM),
        ],
        out_specs=pl.BlockSpec(memory_space=pltpu.MemorySpace.HBM),
        out_shape=jax.ShapeDtypeStruct((B, C, H, W), jnp.float32),
        scratch_shapes=[
            pltpu.VMEM((_NBUF, C, H, W), jnp.float32),
            pltpu.VMEM((_NBUF, C, H, W), jnp.float32),
            pltpu.SemaphoreType.DMA((_NBUF,)),
            pltpu.SemaphoreType.DMA((_NBUF,)),
        ],
    )(cat_logits, gammas, betas_aug, depth_logits.reshape(1, k + 1), ua, ud,
      x4)
    return out.reshape(input.shape)


# speedup vs baseline: 1.1433x; 1.0013x over previous
import functools

import jax
import jax.numpy as jnp
from jax.experimental import pallas as pl
from jax.experimental.pallas import tpu as pltpu

_NBUF = 8


def _first_argmax(z, axis):
    zmax = jnp.max(z, axis=axis, keepdims=True)
    iota = jax.lax.broadcasted_iota(jnp.int32, z.shape, axis)
    big = jnp.int32(z.shape[axis])
    return jnp.min(jnp.where(z >= zmax, iota, big), axis=axis, keepdims=True)


def _routing(cat_ref, gam_ref, bet_ref, dep_ref, ua_ref, ud_ref, k):
    g = -jnp.log(-jnp.log(ua_ref[...]))
    z = cat_ref[...][None, :, :] + g
    idx = _first_argmax(z, axis=2)
    iota = jax.lax.broadcasted_iota(jnp.int32, z.shape, 2)
    oh = (iota == idx).astype(jnp.float32)
    s = jnp.sum(oh * gam_ref[...][None, :, :], axis=2)
    t = jnp.sum(oh * bet_ref[...][None, :, :], axis=2)

    gd = -jnp.log(-jnp.log(ud_ref[...]))
    zd = dep_ref[...] + gd
    m = _first_argmax(zd, axis=1)

    A = jnp.ones_like(m, dtype=jnp.float32)
    C = jnp.zeros_like(A)
    S = jnp.where(m == 0, A, 0.0)
    T = jnp.zeros_like(A)
    for i in range(k):
        si = s[:, i:i + 1]
        ti = t[:, i:i + 1]
        A = si * A
        C = si * C + ti
        S = jnp.where(m == i + 1, A, S)
        T = jnp.where(m == i + 1, C, T)
    return S, T


def _body(cat_ref, gam_ref, bet_ref, dep_ref, ua_ref, ud_ref, x_hbm, o_hbm,
          ibuf, obuf, in_sems, out_sems, *, k, B):
    for j in range(_NBUF):
        pltpu.make_async_copy(x_hbm.at[j], ibuf.at[j], in_sems.at[j]).start()

    S, T = _routing(cat_ref, gam_ref, bet_ref, dep_ref, ua_ref, ud_ref, k)

    for i in range(B):
        slot = i % _NBUF
        pltpu.make_async_copy(x_hbm.at[i], ibuf.at[slot], in_sems.at[slot]).wait()
        if i >= _NBUF:
            pltpu.make_async_copy(
                obuf.at[slot], o_hbm.at[i - _NBUF], out_sems.at[slot]).wait()
        obuf[slot] = S[i, 0] * ibuf[slot] + T[i, 0]
        pltpu.make_async_copy(obuf.at[slot], o_hbm.at[i], out_sems.at[slot]).start()
        nxt = i + _NBUF
        if nxt < B:
            pltpu.make_async_copy(
                x_hbm.at[nxt], ibuf.at[slot], in_sems.at[slot]).start()
    for i in range(B - _NBUF, B):
        slot = i % _NBUF
        pltpu.make_async_copy(obuf.at[slot], o_hbm.at[i], out_sems.at[slot]).wait()


def kernel(input, cat_logits, gammas, betas_aug, depth_logits):
    B = input.shape[0]
    k, d = cat_logits.shape
    C = input.shape[1]
    H, W = 392, 128
    x4 = input.reshape(B, C, H, W)

    key = jax.random.key(42)
    k_aug, k_depth = jax.random.split(key)
    ua = jax.random.uniform(k_aug, (B, k, d), minval=1e-6, maxval=1.0 - 1e-6)
    ud = jax.random.uniform(k_depth, (B, k + 1), minval=1e-6, maxval=1.0 - 1e-6)

    out = pl.pallas_call(
        functools.partial(_body, k=k, B=B),
        in_specs=[
            pl.BlockSpec(memory_space=pltpu.MemorySpace.VMEM),
            pl.BlockSpec(memory_space=pltpu.MemorySpace.VMEM),
            pl.BlockSpec(memory_space=pltpu.MemorySpace.VMEM),
            pl.BlockSpec(memory_space=pltpu.MemorySpace.VMEM),
            pl.BlockSpec(memory_space=pltpu.MemorySpace.VMEM),
            pl.BlockSpec(memory_space=pltpu.MemorySpace.VMEM),
            pl.BlockSpec(memory_space=pltpu.MemorySpace.HBM),
        ],
        out_specs=pl.BlockSpec(memory_space=pltpu.MemorySpace.HBM),
        out_shape=jax.ShapeDtypeStruct((B, C, H, W), jnp.float32),
        compiler_params=pltpu.CompilerParams(
            allow_input_fusion=[False] * 6 + [True]),
        scratch_shapes=[
            pltpu.VMEM((_NBUF, C, H, W), jnp.float32),
            pltpu.VMEM((_NBUF, C, H, W), jnp.float32),
            pltpu.SemaphoreType.DMA((_NBUF,)),
            pltpu.SemaphoreType.DMA((_NBUF,)),
        ],
    )(cat_logits, gammas, betas_aug, depth_logits.reshape(1, k + 1), ua, ud,
      x4)
    return out.reshape(input.shape)
